# Initial kernel scaffold; baseline (speedup 1.0000x reference)
import jax
import jax.numpy as jnp
from jax.experimental import pallas as pl


def kernel(x1, x2, x3, W1, b1, W2, b2, W3, b3):
    raise NotImplementedError("write your pallas kernel here")


# speedup vs baseline: 13.5825x; 13.5825x over previous
import jax
import jax.numpy as jnp
from jax.experimental import pallas as pl
from jax.experimental.pallas import tpu as pltpu

ETA = 1.0
ALPHA = 0.08
BETA = 0.01
TAU = 10.0
PROB_THRESHOLD = 0.8


def _row_softmax(logits):
    m = jnp.max(logits, axis=-1, keepdims=True)
    e = jnp.exp(logits - m)
    return e / jnp.sum(e, axis=-1, keepdims=True)


def _adjacency(D):
    n = D.shape[0]
    Dmax = jnp.max(D, axis=-1, keepdims=True)
    mu = jnp.mean(D)
    sigma = jnp.sqrt(jnp.sum((D - mu) ** 2) / (D.size - 1))
    p = _row_softmax(-D / TAU)
    ent = -jnp.sum(p * jnp.log(p + 1e-8), axis=-1, keepdims=True)
    decay = jnp.exp(-ETA * ent)
    T = mu + ALPHA * sigma + BETA * (1.0 - D / Dmax) * decay
    pad = 8 * ((n + 7) // 8) - n
    DT = jnp.concatenate([D.T, jnp.zeros((pad, n), jnp.float32)], axis=0)
    pT = jnp.concatenate([p.T, jnp.zeros((pad, n), jnp.float32)], axis=0)
    cmp = DT[:, :, None] < D[None, :, :]
    S = jnp.sum(jnp.where(cmp, pT[:, :, None], 0.0), axis=0)
    return ((S + p) <= PROB_THRESHOLD) & (D < T)


def _euclid_D(x):
    xT = x.T
    diff = xT[:, :, None] - xT[:, None, :]
    return jnp.sqrt(jnp.sum(diff * diff, axis=0))


def _tri_kernel(x1_ref, x2_ref, x3_ref, w1_ref, b1_ref, w2_ref, b2_ref,
                w3_ref, b3_ref, out_ref):
    x = [x1_ref[0], x2_ref[0], x3_ref[0]]
    ws = [w1_ref[...], w2_ref[...], w3_ref[...]]
    bs = [b1_ref[...], b2_ref[...], b3_ref[...]]
    f32 = jnp.float32
    dn_nt = (((1,), (1,)), ((), ()))
    dn_nn = (((1,), (0,)), ((), ()))

    def dot(a, b, dn):
        return jax.lax.dot_general(a, b, dn, preferred_element_type=f32)

    y = [dot(x[i], ws[i], dn_nt) + bs[i] for i in range(3)]
    xn = []
    for i in range(3):
        nrm = jnp.sqrt(jnp.sum(x[i] * x[i], axis=-1, keepdims=True))
        xn.append(x[i] / jnp.maximum(nrm, 1e-12))

    pairs = [(0, 0), (1, 1), (2, 2),
             (0, 1), (1, 0), (0, 2), (2, 0), (1, 2), (2, 1)]
    acc = [None, None, None]
    for i, j in pairs:
        if i == j:
            D = _euclid_D(x[i])
        else:
            D = 1.0 - dot(xn[i], xn[j], dn_nt)
        A = _adjacency(D)
        att = dot(y[i], y[j], dn_nt)
        sm = _row_softmax(jnp.where(A, att, 0.0))
        acc[j] = sm if acc[j] is None else acc[j] + sm

    out = (dot(acc[0], x[0], dn_nn) + dot(acc[1], x[1], dn_nn)
           + dot(acc[2], x[2], dn_nn))
    out_ref[0] = out


def kernel(x1, x2, x3, W1, b1, W2, b2, W3, b3):
    b, c, h, w = x1.shape
    n = h * w
    x1f = x1.reshape(b, c, n).transpose(0, 2, 1)
    x2f = x2.reshape(b, c, n).transpose(0, 2, 1)
    x3f = x3.reshape(b, c, n).transpose(0, 2, 1)
    b1r = b1.reshape(1, c)
    b2r = b2.reshape(1, c)
    b3r = b3.reshape(1, c)

    x_spec = pl.BlockSpec((1, n, c), lambda i: (i, 0, 0))
    w_spec = pl.BlockSpec((c, c), lambda i: (0, 0))
    b_spec = pl.BlockSpec((1, c), lambda i: (0, 0))

    out = pl.pallas_call(
        _tri_kernel,
        grid=(b,),
        in_specs=[x_spec, x_spec, x_spec,
                  w_spec, b_spec, w_spec, b_spec, w_spec, b_spec],
        out_specs=pl.BlockSpec((1, n, c), lambda i: (i, 0, 0)),
        out_shape=jax.ShapeDtypeStruct((b, n, c), jnp.float32),
        compiler_params=pltpu.CompilerParams(
            dimension_semantics=("arbitrary",)),
    )(x1f, x2f, x3f, W1, b1r, W2, b2r, W3, b3r)
    return out.transpose(0, 2, 1).reshape(b, c, h, w)


# speedup vs baseline: 49.5694x; 3.6495x over previous
import jax
import jax.numpy as jnp
from jax.experimental import pallas as pl
from jax.experimental.pallas import tpu as pltpu

ETA = 1.0
ALPHA = 0.08
BETA = 0.01
TAU = 10.0
PROB_THRESHOLD = 0.8


def _colwise_softmax(logits):
    m = jnp.max(logits, axis=0, keepdims=True)
    e = jnp.exp(logits - m)
    return e / jnp.sum(e, axis=0, keepdims=True)


def _adjacency_T(DT):
    Dmax = jnp.max(DT, axis=0, keepdims=True)
    mu = jnp.mean(DT)
    sigma = jnp.sqrt(jnp.sum((DT - mu) ** 2) / (DT.size - 1))
    pT = _colwise_softmax(-DT / TAU)
    ent = -jnp.sum(pT * jnp.log(pT + 1e-8), axis=0, keepdims=True)
    decay = jnp.exp(-ETA * ent)
    TT = mu + ALPHA * sigma + BETA * (1.0 - DT / Dmax) * decay

    bits = jax.lax.bitcast_convert_type(DT, jnp.int32)
    Di = jnp.where(bits >= 0, bits, bits ^ jnp.int32(0x7FFFFFFF))
    lo = jnp.min(Di, axis=0, keepdims=True) - 1
    hi = jnp.max(Di, axis=0, keepdims=True)

    def body(_, carry):
        lo, hi = carry
        mid = (lo >> 1) + (hi >> 1) + (lo & hi & 1)
        s = jnp.sum(jnp.where(Di <= mid, pT, 0.0), axis=0, keepdims=True)
        ok = s <= PROB_THRESHOLD
        return jnp.where(ok, mid, lo), jnp.where(ok, hi, mid)

    lo, hi = jax.lax.fori_loop(0, 32, body, (lo, hi))
    return (Di <= lo) & (DT < TT)


def _euclid_D(x):
    xT = x.T
    diff = xT[:, :, None] - xT[:, None, :]
    return jnp.sqrt(jnp.sum(diff * diff, axis=0))


def _tri_kernel(x1_ref, x2_ref, x3_ref, w1_ref, b1_ref, w2_ref, b2_ref,
                w3_ref, b3_ref, out_ref):
    x = [x1_ref[0], x2_ref[0], x3_ref[0]]
    ws = [w1_ref[...], w2_ref[...], w3_ref[...]]
    bs = [b1_ref[...], b2_ref[...], b3_ref[...]]
    f32 = jnp.float32
    dn_nt = (((1,), (1,)), ((), ()))
    dn_tn = (((0,), (0,)), ((), ()))

    def dot(a, b, dn):
        return jax.lax.dot_general(a, b, dn, preferred_element_type=f32)

    y = [dot(x[i], ws[i], dn_nt) + bs[i] for i in range(3)]
    xn = []
    for i in range(3):
        nrm = jnp.sqrt(jnp.sum(x[i] * x[i], axis=-1, keepdims=True))
        xn.append(x[i] / jnp.maximum(nrm, 1e-12))

    pairs = [(0, 0), (1, 1), (2, 2),
             (0, 1), (1, 0), (0, 2), (2, 0), (1, 2), (2, 1)]
    acc = [None, None, None]
    for i, j in pairs:
        if i == j:
            DT = _euclid_D(x[i])
        else:
            DT = 1.0 - dot(xn[j], xn[i], dn_nt)
        AT = _adjacency_T(DT)
        attT = dot(y[j], y[i], dn_nt)
        smT = _colwise_softmax(jnp.where(AT, attT, 0.0))
        acc[j] = smT if acc[j] is None else acc[j] + smT

    out = (dot(acc[0], x[0], dn_tn) + dot(acc[1], x[1], dn_tn)
           + dot(acc[2], x[2], dn_tn))
    out_ref[0] = out


def kernel(x1, x2, x3, W1, b1, W2, b2, W3, b3):
    b, c, h, w = x1.shape
    n = h * w
    x1f = x1.reshape(b, c, n).transpose(0, 2, 1)
    x2f = x2.reshape(b, c, n).transpose(0, 2, 1)
    x3f = x3.reshape(b, c, n).transpose(0, 2, 1)
    b1r = b1.reshape(1, c)
    b2r = b2.reshape(1, c)
    b3r = b3.reshape(1, c)

    x_spec = pl.BlockSpec((1, n, c), lambda i: (i, 0, 0))
    w_spec = pl.BlockSpec((c, c), lambda i: (0, 0))
    b_spec = pl.BlockSpec((1, c), lambda i: (0, 0))

    out = pl.pallas_call(
        _tri_kernel,
        grid=(b,),
        in_specs=[x_spec, x_spec, x_spec,
                  w_spec, b_spec, w_spec, b_spec, w_spec, b_spec],
        out_specs=pl.BlockSpec((1, n, c), lambda i: (i, 0, 0)),
        out_shape=jax.ShapeDtypeStruct((b, n, c), jnp.float32),
        compiler_params=pltpu.CompilerParams(
            dimension_semantics=("arbitrary",)),
    )(x1f, x2f, x3f, W1, b1r, W2, b2r, W3, b3r)
    return out.transpose(0, 2, 1).reshape(b, c, h, w)


# speedup vs baseline: 81.2774x; 1.6397x over previous
import jax
import jax.numpy as jnp
from jax.experimental import pallas as pl
from jax.experimental.pallas import tpu as pltpu

ETA = 1.0
ALPHA = 0.08
BETA = 0.01
TAU = 10.0
PROB_THRESHOLD = 0.8


def _colwise_softmax(logits):
    m = jnp.max(logits, axis=0, keepdims=True)
    e = jnp.exp(logits - m)
    return e / jnp.sum(e, axis=0, keepdims=True)


def _adjacency_prep(DT):
    Dmax = jnp.max(DT, axis=0, keepdims=True)
    mu = jnp.mean(DT)
    sigma = jnp.sqrt(jnp.sum((DT - mu) ** 2) / (DT.size - 1))
    pT = _colwise_softmax(-DT / TAU)
    ent = -jnp.sum(pT * jnp.log(pT + 1e-8), axis=0, keepdims=True)
    decay = jnp.exp(-ETA * ent)
    TT = mu + ALPHA * sigma + BETA * (1.0 - DT / Dmax) * decay
    bits = jax.lax.bitcast_convert_type(DT, jnp.int32)
    Di = jnp.where(bits >= 0, bits, bits ^ jnp.int32(0x7FFFFFFF))
    return Di, pT, TT


def _bisect_all(Dis, pTs):
    los = tuple(jnp.min(Di, axis=0, keepdims=True) - 1 for Di in Dis)
    his = tuple(jnp.max(Di, axis=0, keepdims=True) for Di in Dis)

    def body(_, carry):
        los, his = carry
        nlo, nhi = [], []
        for Di, pT, lo, hi in zip(Dis, pTs, los, his):
            mid = (lo >> 1) + (hi >> 1) + (lo & hi & 1)
            s = jnp.sum(jnp.where(Di <= mid, pT, 0.0), axis=0, keepdims=True)
            ok = s <= PROB_THRESHOLD
            nlo.append(jnp.where(ok, mid, lo))
            nhi.append(jnp.where(ok, hi, mid))
        return tuple(nlo), tuple(nhi)

    los, his = jax.lax.fori_loop(0, 32, body, (los, his))
    return los


def _euclid_D(x):
    xT = x.T
    diff = xT[:, :, None] - xT[:, None, :]
    return jnp.sqrt(jnp.sum(diff * diff, axis=0))


def _tri_kernel(x1_ref, x2_ref, x3_ref, w1_ref, b1_ref, w2_ref, b2_ref,
                w3_ref, b3_ref, out_ref):
    x = [x1_ref[0], x2_ref[0], x3_ref[0]]
    ws = [w1_ref[...], w2_ref[...], w3_ref[...]]
    bs = [b1_ref[...], b2_ref[...], b3_ref[...]]
    f32 = jnp.float32
    dn_nt = (((1,), (1,)), ((), ()))
    dn_tn = (((0,), (0,)), ((), ()))

    def dot(a, b, dn):
        return jax.lax.dot_general(a, b, dn, preferred_element_type=f32)

    y = [dot(x[i], ws[i], dn_nt) + bs[i] for i in range(3)]
    xn = []
    for i in range(3):
        nrm = jnp.sqrt(jnp.sum(x[i] * x[i], axis=-1, keepdims=True))
        xn.append(x[i] / jnp.maximum(nrm, 1e-12))

    pairs = [(0, 0), (1, 1), (2, 2),
             (0, 1), (1, 0), (0, 2), (2, 0), (1, 2), (2, 1)]
    Dis, pTs, TTlts = [], [], []
    for i, j in pairs:
        if i == j:
            DT = _euclid_D(x[i])
        else:
            DT = 1.0 - dot(xn[j], xn[i], dn_nt)
        Di, pT, TT = _adjacency_prep(DT)
        Dis.append(Di)
        pTs.append(pT)
        TTlts.append(DT < TT)
    los = _bisect_all(Dis, pTs)

    acc = [None, None, None]
    for (i, j), Di, lo, TTlt in zip(pairs, Dis, los, TTlts):
        AT = (Di <= lo) & TTlt
        attT = dot(y[j], y[i], dn_nt)
        smT = _colwise_softmax(jnp.where(AT, attT, 0.0))
        acc[j] = smT if acc[j] is None else acc[j] + smT

    out = (dot(acc[0], x[0], dn_tn) + dot(acc[1], x[1], dn_tn)
           + dot(acc[2], x[2], dn_tn))
    out_ref[0] = out


def kernel(x1, x2, x3, W1, b1, W2, b2, W3, b3):
    b, c, h, w = x1.shape
    n = h * w
    x1f = x1.reshape(b, c, n).transpose(0, 2, 1)
    x2f = x2.reshape(b, c, n).transpose(0, 2, 1)
    x3f = x3.reshape(b, c, n).transpose(0, 2, 1)
    b1r = b1.reshape(1, c)
    b2r = b2.reshape(1, c)
    b3r = b3.reshape(1, c)

    x_spec = pl.BlockSpec((1, n, c), lambda i: (i, 0, 0))
    w_spec = pl.BlockSpec((c, c), lambda i: (0, 0))
    b_spec = pl.BlockSpec((1, c), lambda i: (0, 0))

    out = pl.pallas_call(
        _tri_kernel,
        grid=(b,),
        in_specs=[x_spec, x_spec, x_spec,
                  w_spec, b_spec, w_spec, b_spec, w_spec, b_spec],
        out_specs=pl.BlockSpec((1, n, c), lambda i: (i, 0, 0)),
        out_shape=jax.ShapeDtypeStruct((b, n, c), jnp.float32),
        compiler_params=pltpu.CompilerParams(
            dimension_semantics=("arbitrary",)),
    )(x1f, x2f, x3f, W1, b1r, W2, b2r, W3, b3r)
    return out.transpose(0, 2, 1).reshape(b, c, h, w)


# speedup vs baseline: 84.0021x; 1.0335x over previous
import jax
import jax.numpy as jnp
from jax.experimental import pallas as pl
from jax.experimental.pallas import tpu as pltpu

ETA = 1.0
ALPHA = 0.08
BETA = 0.01
TAU = 10.0
PROB_THRESHOLD = 0.8


def _colwise_softmax(logits):
    m = jnp.max(logits, axis=0, keepdims=True)
    e = jnp.exp(logits - m)
    return e / jnp.sum(e, axis=0, keepdims=True)


def _adjacency_prep(DT):
    Dmax = jnp.max(DT, axis=0, keepdims=True)
    mu = jnp.mean(DT)
    sigma = jnp.sqrt(jnp.sum((DT - mu) ** 2) / (DT.size - 1))
    L = -DT / TAU
    m = jnp.max(L, axis=0, keepdims=True)
    e = jnp.exp(L - m)
    Z = jnp.sum(e, axis=0, keepdims=True)
    pT = e / Z
    ent = jnp.sum(pT * (DT / TAU), axis=0, keepdims=True) + m + jnp.log(Z)
    decay = jnp.exp(-ETA * ent)
    TT = mu + ALPHA * sigma + BETA * (1.0 - DT / Dmax) * decay
    bits = jax.lax.bitcast_convert_type(DT, jnp.int32)
    Di = jnp.where(bits >= 0, bits, bits ^ jnp.int32(0x7FFFFFFF))
    return Di, pT, TT


def _bisect_all(Dis, pTs):
    los = tuple(jnp.min(Di, axis=0, keepdims=True) - 1 for Di in Dis)
    his = tuple(jnp.max(Di, axis=0, keepdims=True) for Di in Dis)

    def body(_, carry):
        los, his = carry
        for _ in range(4):
            nlo, nhi = [], []
            for Di, pT, lo, hi in zip(Dis, pTs, los, his):
                mid = (lo >> 1) + (hi >> 1) + (lo & hi & 1)
                s = jnp.sum(jnp.where(Di <= mid, pT, 0.0), axis=0,
                            keepdims=True)
                ok = s <= PROB_THRESHOLD
                nlo.append(jnp.where(ok, mid, lo))
                nhi.append(jnp.where(ok, hi, mid))
            los, his = tuple(nlo), tuple(nhi)
        return los, his

    los, his = jax.lax.fori_loop(0, 8, body, (los, his))
    return los


def _euclid_D(x):
    xT = x.T
    diff = xT[:, :, None] - xT[:, None, :]
    return jnp.sqrt(jnp.sum(diff * diff, axis=0))


def _tri_kernel(x1_ref, x2_ref, x3_ref, w1_ref, b1_ref, w2_ref, b2_ref,
                w3_ref, b3_ref, out_ref):
    x = [x1_ref[0], x2_ref[0], x3_ref[0]]
    ws = [w1_ref[...], w2_ref[...], w3_ref[...]]
    bs = [b1_ref[...], b2_ref[...], b3_ref[...]]
    f32 = jnp.float32
    dn_nt = (((1,), (1,)), ((), ()))
    dn_tn = (((0,), (0,)), ((), ()))

    def dot(a, b, dn):
        return jax.lax.dot_general(a, b, dn, preferred_element_type=f32)

    y = [dot(x[i], ws[i], dn_nt) + bs[i] for i in range(3)]
    xn = []
    for i in range(3):
        nrm = jnp.sqrt(jnp.sum(x[i] * x[i], axis=-1, keepdims=True))
        xn.append(x[i] / jnp.maximum(nrm, 1e-12))

    pairs = [(0, 0), (1, 1), (2, 2),
             (0, 1), (1, 0), (0, 2), (2, 0), (1, 2), (2, 1)]
    Dis, pTs, TTlts = [], [], []
    for i, j in pairs:
        if i == j:
            DT = _euclid_D(x[i])
        else:
            DT = 1.0 - dot(xn[j], xn[i], dn_nt)
        Di, pT, TT = _adjacency_prep(DT)
        Dis.append(Di)
        pTs.append(pT)
        TTlts.append(DT < TT)
    los = _bisect_all(Dis, pTs)

    acc = [None, None, None]
    for (i, j), Di, lo, TTlt in zip(pairs, Dis, los, TTlts):
        AT = (Di <= lo) & TTlt
        attT = dot(y[j], y[i], dn_nt)
        smT = _colwise_softmax(jnp.where(AT, attT, 0.0))
        acc[j] = smT if acc[j] is None else acc[j] + smT

    out = (dot(acc[0], x[0], dn_tn) + dot(acc[1], x[1], dn_tn)
           + dot(acc[2], x[2], dn_tn))
    out_ref[0] = out


def kernel(x1, x2, x3, W1, b1, W2, b2, W3, b3):
    b, c, h, w = x1.shape
    n = h * w
    x1f = x1.reshape(b, c, n).transpose(0, 2, 1)
    x2f = x2.reshape(b, c, n).transpose(0, 2, 1)
    x3f = x3.reshape(b, c, n).transpose(0, 2, 1)
    b1r = b1.reshape(1, c)
    b2r = b2.reshape(1, c)
    b3r = b3.reshape(1, c)

    x_spec = pl.BlockSpec((1, n, c), lambda i: (i, 0, 0))
    w_spec = pl.BlockSpec((c, c), lambda i: (0, 0))
    b_spec = pl.BlockSpec((1, c), lambda i: (0, 0))

    out = pl.pallas_call(
        _tri_kernel,
        grid=(b,),
        in_specs=[x_spec, x_spec, x_spec,
                  w_spec, b_spec, w_spec, b_spec, w_spec, b_spec],
        out_specs=pl.BlockSpec((1, n, c), lambda i: (i, 0, 0)),
        out_shape=jax.ShapeDtypeStruct((b, n, c), jnp.float32),
        compiler_params=pltpu.CompilerParams(
            dimension_semantics=("arbitrary",)),
    )(x1f, x2f, x3f, W1, b1r, W2, b2r, W3, b3r)
    return out.transpose(0, 2, 1).reshape(b, c, h, w)
